# baseline (speedup 1.0000x reference)
import jax
import jax.numpy as jnp
from jax.experimental import pallas as pl

_B, _N, _DIN, _H, _DOUT = 4, 2048, 128, 64, 16
_F = _N * 2 * _DOUT


def _gcn_body(x_ref, s_ref, w1_ref, w2_ref, out_ref):
    s = s_ref[0]
    xw = jnp.dot(x_ref[0], w1_ref[...], preferred_element_type=jnp.float32)
    h1 = jnp.maximum(jnp.dot(s, xw, preferred_element_type=jnp.float32), 0.0)
    hw = jnp.dot(h1, w2_ref[...], preferred_element_type=jnp.float32)
    h2 = jnp.maximum(jnp.dot(s, hw, preferred_element_type=jnp.float32), 0.0)
    out_ref[0] = h2


def _readout_body(f_ref, wr1_ref, br1_ref, wr2_ref, br2_ref, out_ref):
    o1 = jnp.dot(f_ref[...], wr1_ref[...], preferred_element_type=jnp.float32)
    o1 = jnp.maximum(o1 + br1_ref[...], 0.0)
    o = jnp.dot(o1, wr2_ref[...], preferred_element_type=jnp.float32)
    o = o + br2_ref[...]
    m = jnp.max(o, axis=-1, keepdims=True)
    lse = m + jnp.log(jnp.sum(jnp.exp(o - m), axis=-1, keepdims=True))
    out_ref[...] = o - lse


@jax.jit
def kernel(x, support, W1, W2, Wr1, br1, Wr2, br2):
    h2 = pl.pallas_call(
        _gcn_body,
        grid=(_B,),
        in_specs=[
            pl.BlockSpec((1, _N, _DIN), lambda b: (b, 0, 0)),
            pl.BlockSpec((1, _N, _N), lambda b: (b, 0, 0)),
            pl.BlockSpec((_DIN, _H), lambda b: (0, 0)),
            pl.BlockSpec((_H, 2 * _DOUT), lambda b: (0, 0)),
        ],
        out_specs=pl.BlockSpec((1, _N, 2 * _DOUT), lambda b: (b, 0, 0)),
        out_shape=jax.ShapeDtypeStruct((_B, _N, 2 * _DOUT), jnp.float32),
    )(x, support, W1, W2)

    f = h2.reshape(_B, _F)
    out = pl.pallas_call(
        _readout_body,
        in_specs=[
            pl.BlockSpec((_B, _F), lambda: (0, 0)),
            pl.BlockSpec((_F, 64), lambda: (0, 0)),
            pl.BlockSpec((1, 64), lambda: (0, 0)),
            pl.BlockSpec((64, _DOUT), lambda: (0, 0)),
            pl.BlockSpec((1, _DOUT), lambda: (0, 0)),
        ],
        out_specs=pl.BlockSpec((_B, _DOUT), lambda: (0, 0)),
        out_shape=jax.ShapeDtypeStruct((_B, _DOUT), jnp.float32),
    )(f, Wr1, br1.reshape(1, 64), Wr2, br2.reshape(1, _DOUT))
    return out
